# baseline (speedup 1.0000x reference)
import jax
import jax.numpy as jnp
from jax import lax
from jax.experimental import pallas as pl
from jax.experimental.pallas import tpu as pltpu
from jax.experimental.pallas import tpu_sc as plsc

_CH = 64


def _min_body(in_ref, out_ref, acc_ref):
    b = pl.program_id(0)
    m = jnp.min(in_ref[...])

    @pl.when(b == 0)
    def _init():
        acc_ref[0] = m

    @pl.when(b > 0)
    def _acc():
        acc_ref[0] = jnp.minimum(acc_ref[0], m)

    @pl.when(b == pl.num_programs(0) - 1)
    def _out():
        out_ref[0, 0] = acc_ref[0]


def _global_min(inputs):
    B, S, D = inputs.shape
    return pl.pallas_call(
        _min_body,
        grid=(B,),
        in_specs=[pl.BlockSpec((1, S, D), lambda b: (b, 0, 0))],
        out_specs=pl.BlockSpec(memory_space=pltpu.SMEM),
        out_shape=jax.ShapeDtypeStruct((1, 1), jnp.float32),
        scratch_shapes=[pltpu.SMEM((1,), jnp.float32)],
    )(inputs)


def _sc_copy(inputs):
    B, S, D = inputs.shape
    info = plsc.get_sparse_core_info()
    nw = info.num_cores * info.num_subcores
    rows_per_w = (B * S) // nw
    n_chunks = rows_per_w // _CH
    mesh = plsc.VectorSubcoreMesh(core_axis_name="c", subcore_axis_name="s")

    def body(in_hbm, out_hbm, buf, sem_in, sem_out):
        c = lax.axis_index("c")
        s = lax.axis_index("s")
        wid = s * info.num_cores + c
        b = wid // (S // rows_per_w)
        row0 = (wid % (S // rows_per_w)) * rows_per_w

        def step(i, carry):
            r = row0 + i * _CH
            pltpu.make_async_copy(
                in_hbm.at[b, pl.ds(r, _CH), :], buf, sem_in).start()
            pltpu.make_async_copy(
                in_hbm.at[b, pl.ds(r, _CH), :], buf, sem_in).wait()
            pltpu.make_async_copy(
                buf, out_hbm.at[b, pl.ds(r, _CH), :], sem_out).start()
            pltpu.make_async_copy(
                buf, out_hbm.at[b, pl.ds(r, _CH), :], sem_out).wait()
            return carry

        lax.fori_loop(0, n_chunks, step, 0)

    return pl.kernel(
        body,
        out_type=jax.ShapeDtypeStruct((B, 2 * S, D), inputs.dtype),
        mesh=mesh,
        scratch_types=[
            pltpu.VMEM((_CH, D), jnp.float32),
            pltpu.SemaphoreType.DMA,
            pltpu.SemaphoreType.DMA,
        ],
    )(inputs)


def _fill_body(_, min_ref, out_ref):
    out_ref[...] = jnp.full(out_ref.shape, min_ref[0, 0] - 0.01,
                            out_ref.dtype)


def _fill_pad(out_buf, minv):
    B, T, D = out_buf.shape
    S = T // 2
    return pl.pallas_call(
        _fill_body,
        grid=(B,),
        in_specs=[
            pl.BlockSpec(memory_space=pl.ANY),
            pl.BlockSpec(memory_space=pltpu.SMEM),
        ],
        out_specs=pl.BlockSpec((1, S, D), lambda b: (b, 1, 0)),
        out_shape=jax.ShapeDtypeStruct((B, T, D), out_buf.dtype),
        input_output_aliases={0: 0},
    )(out_buf, minv)


def kernel(inputs):
    minv = _global_min(inputs)
    out = _sc_copy(inputs)
    return _fill_pad(out, minv)


# speedup vs baseline: 1.0032x; 1.0032x over previous
import jax
import jax.numpy as jnp
from jax import lax
from jax.experimental import pallas as pl
from jax.experimental.pallas import tpu as pltpu
from jax.experimental.pallas import tpu_sc as plsc

_CH = 64


def _min_body(in_ref, out_ref, acc_ref):
    b = pl.program_id(0)
    m = jnp.min(in_ref[...])

    @pl.when(b == 0)
    def _init():
        acc_ref[0] = m

    @pl.when(b > 0)
    def _acc():
        acc_ref[0] = jnp.minimum(acc_ref[0], m)

    @pl.when(b == pl.num_programs(0) - 1)
    def _out():
        out_ref[0, 0] = acc_ref[0]


def _global_min(inputs):
    B, S, D = inputs.shape
    return pl.pallas_call(
        _min_body,
        grid=(B,),
        in_specs=[pl.BlockSpec((1, S, D), lambda b: (b, 0, 0))],
        out_specs=pl.BlockSpec(memory_space=pltpu.SMEM),
        out_shape=jax.ShapeDtypeStruct((1, 1), jnp.float32),
        scratch_shapes=[pltpu.SMEM((1,), jnp.float32)],
    )(inputs)


def _sc_copy(inputs):
    B, S, D = inputs.shape
    info = plsc.get_sparse_core_info()
    nw = info.num_cores * info.num_subcores
    rows_per_w = (B * S) // nw
    n_chunks = rows_per_w // _CH
    mesh = plsc.VectorSubcoreMesh(core_axis_name="c", subcore_axis_name="s")

    def body(in_hbm, out_hbm, buf0, buf1, si0, si1, so0, so1):
        c = lax.axis_index("c")
        s = lax.axis_index("s")
        wid = s * info.num_cores + c
        b = wid // (S // rows_per_w)
        row0 = (wid % (S // rows_per_w)) * rows_per_w
        bufs = (buf0, buf1)
        sis = (si0, si1)
        sos = (so0, so1)

        def in_cp(i):
            r = row0 + i * _CH
            return pltpu.make_async_copy(
                in_hbm.at[b, pl.ds(r, _CH), :], bufs[i % 2], sis[i % 2])

        def out_cp(i):
            r = row0 + i * _CH
            return pltpu.make_async_copy(
                bufs[i % 2], out_hbm.at[b, pl.ds(r, _CH), :], sos[i % 2])

        in_cp(0).start()
        for i in range(n_chunks):
            if i >= 1:
                out_cp(i - 1).wait()
            if i + 1 < n_chunks:
                in_cp(i + 1).start()
            in_cp(i).wait()
            out_cp(i).start()
        out_cp(n_chunks - 1).wait()

    return pl.kernel(
        body,
        out_type=jax.ShapeDtypeStruct((B, 2 * S, D), inputs.dtype),
        mesh=mesh,
        scratch_types=[
            pltpu.VMEM((_CH, D), jnp.float32),
            pltpu.VMEM((_CH, D), jnp.float32),
            pltpu.SemaphoreType.DMA,
            pltpu.SemaphoreType.DMA,
            pltpu.SemaphoreType.DMA,
            pltpu.SemaphoreType.DMA,
        ],
    )(inputs)


def _fill_body(_, min_ref, out_ref):
    out_ref[...] = jnp.full(out_ref.shape, min_ref[0, 0] - 0.01,
                            out_ref.dtype)


def _fill_pad(out_buf, minv):
    B, T, D = out_buf.shape
    S = T // 2
    return pl.pallas_call(
        _fill_body,
        grid=(B,),
        in_specs=[
            pl.BlockSpec(memory_space=pl.ANY),
            pl.BlockSpec(memory_space=pltpu.SMEM),
        ],
        out_specs=pl.BlockSpec((1, S, D), lambda b: (b, 1, 0)),
        out_shape=jax.ShapeDtypeStruct((B, T, D), out_buf.dtype),
        input_output_aliases={0: 0},
    )(out_buf, minv)


def kernel(inputs):
    minv = _global_min(inputs)
    out = _sc_copy(inputs)
    return _fill_pad(out, minv)


# speedup vs baseline: 1.5163x; 1.5114x over previous
import jax
import jax.numpy as jnp
from jax.experimental import pallas as pl
from jax.experimental.pallas import tpu as pltpu

_BS = 2048


def _pad_kernel(in_ref, out_ref, min_ref):
    p = pl.program_id(0)
    b = pl.program_id(1)
    s = pl.program_id(2)

    @pl.when(p == 0)
    def _copy_and_reduce():
        x = in_ref[...]
        out_ref[...] = x
        m = jnp.min(x)
        first = (b == 0) & (s == 0)

        @pl.when(first)
        def _init():
            min_ref[0] = m

        @pl.when(jnp.logical_not(first))
        def _acc():
            min_ref[0] = jnp.minimum(min_ref[0], m)

    @pl.when(p == 1)
    def _fill_pad():
        out_ref[...] = jnp.full(out_ref.shape, min_ref[0] - 0.01,
                                out_ref.dtype)


def kernel(inputs):
    B, S, D = inputs.shape
    nb = S // _BS
    in_spec = pl.BlockSpec(
        (1, _BS, D),
        lambda p, b, s: (jnp.where(p == 0, b, B - 1),
                         jnp.where(p == 0, s, nb - 1), 0),
    )
    out_spec = pl.BlockSpec((1, _BS, D), lambda p, b, s: (b, p * nb + s, 0))
    return pl.pallas_call(
        _pad_kernel,
        grid=(2, B, nb),
        in_specs=[in_spec],
        out_specs=out_spec,
        out_shape=jax.ShapeDtypeStruct((B, 2 * S, D), inputs.dtype),
        scratch_shapes=[pltpu.SMEM((1,), jnp.float32)],
    )(inputs)
